# baseline (device time: 715107 ns/iter reference)
import jax
import jax.numpy as jnp
from jax import lax
from jax.experimental import pallas as pl
from jax.experimental.pallas import tpu as pltpu

N_DEV = 4
MC = 1024
W = 1024
N_CH = 2
N_SLOT = 2
N_GRP = 2


def kernel(x, w_mat, scale_x, scale_w):
    M, _ = x.shape
    _, N = w_mat.shape
    x = x.astype(jnp.float8_e5m2)
    w_mat = w_mat.astype(jnp.float8_e5m2)

    def body(x_ref, w_ref, sx_ref, sw_ref, dummy_ref, out_ref,
             a_bufs, b_bufs, stage, send_sems, recv_sems, credit_sems,
             store_sem):
        del dummy_ref
        d = lax.axis_index("i")
        left = lax.rem(d + N_DEV - 1, N_DEV)
        right = lax.rem(d + 1, N_DEV)

        barrier_sem = pltpu.get_barrier_semaphore()
        for nbr in (left, right):
            pl.semaphore_signal(barrier_sem, inc=1, device_id=(nbr,),
                                device_id_type=pl.DeviceIdType.MESH)
        pl.semaphore_wait(barrier_sem, 2)

        scale = sx_ref[0] * sw_ref[0]
        dst = (right, left)
        ups = (left, right)

        def pnl(g, ch, di):
            return 4 * g + 2 * ch + di

        def pchunk(c, p):
            xa = x_ref[pl.ds(c * MC, MC), :]
            wb = w_ref[:, pl.ds(p * W, W)]
            return jnp.dot(xa, wb, preferred_element_type=jnp.float32)

        def start_hop(ch, h, srcs, need_credit):
            descs = []
            for di in range(2):
                if need_credit:
                    pl.semaphore_wait(credit_sems.at[di, ch], 1)
                rd = pltpu.make_async_remote_copy(
                    src_ref=srcs[di],
                    dst_ref=b_bufs.at[di, ch, h % N_SLOT],
                    send_sem=send_sems.at[di, ch, h % N_SLOT],
                    recv_sem=recv_sems.at[di, ch, h % N_SLOT],
                    device_id=(dst[di],),
                    device_id_type=pl.DeviceIdType.MESH,
                )
                rd.start()
                descs.append(rd)
            return descs

        def send_credits(ch):
            for di in range(2):
                pl.semaphore_signal(credit_sems.at[di, ch], inc=1,
                                    device_id=(ups[di],),
                                    device_id_type=pl.DeviceIdType.MESH)

        def store(val_f32, c, p):
            stage[...] = val_f32
            st = pltpu.make_async_copy(
                stage, out_ref.at[pl.ds(c * MC, MC), pl.ds(p * W, W)],
                store_sem)
            st.start()
            st.wait()

        own = (lax.rem(d + 1, N_DEV), lax.rem(d + N_DEV - 1, N_DEV))

        infl = {}
        for g in range(N_GRP):
            for ch in range(N_CH):
                a_bufs[0, ch] = pchunk(d, pnl(g, ch, 0)).astype(jnp.bfloat16)
                a_bufs[1, ch] = pchunk(d, pnl(g, ch, 1)).astype(jnp.bfloat16)
                infl[ch] = start_hop(
                    ch, 0, (a_bufs.at[0, ch], a_bufs.at[1, ch]),
                    need_credit=(g > 0))

            for h in range(6):
                for ch in range(N_CH):
                    if h < 3:
                        nxt = (
                            pchunk(lax.rem(d + N_DEV - h - 1, N_DEV),
                                   pnl(g, ch, 0)),
                            pchunk(lax.rem(d + h + 1, N_DEV),
                                   pnl(g, ch, 1)),
                        )
                    for rd in infl[ch]:
                        rd.wait()

                    slot = h % N_SLOT
                    if h < 2:
                        for di in range(2):
                            a_bufs[di, ch] = (
                                b_bufs[di, ch, slot] + nxt[di]
                            ).astype(jnp.bfloat16)
                        send_credits(ch)
                        infl[ch] = start_hop(
                            ch, h + 1,
                            (a_bufs.at[0, ch], a_bufs.at[1, ch]),
                            need_credit=(6 * g + h + 1 >= 2))
                    elif h == 2:
                        acts = []
                        for di in range(2):
                            y = (b_bufs[di, ch, slot] + nxt[di]) * scale
                            act = y / (1.0 + jnp.exp(-jnp.clip(y, -60.0,
                                                               60.0)))
                            a_bufs[di, ch] = act.astype(jnp.bfloat16)
                            acts.append(act)
                        send_credits(ch)
                        infl[ch] = start_hop(
                            ch, 3, (a_bufs.at[0, ch], a_bufs.at[1, ch]),
                            need_credit=True)
                        for di in range(2):
                            store(acts[di], own[di], pnl(g, ch, di))
                    else:
                        t = h - 3
                        if h == 4:
                            send_credits(ch)
                        if h == 5 and g < N_GRP - 1:
                            send_credits(ch)
                        if h < 5:
                            infl[ch] = start_hop(
                                ch, h + 1,
                                (b_bufs.at[0, ch, slot],
                                 b_bufs.at[1, ch, slot]),
                                need_credit=True)
                        rows = (lax.rem(d + N_DEV - t, N_DEV),
                                lax.rem(d + t, N_DEV))
                        for di in range(2):
                            store(b_bufs[di, ch, slot].astype(jnp.float32),
                                  rows[di], pnl(g, ch, di))
                        if h == 5 and g < N_GRP - 1:
                            send_credits(ch)

    return pl.pallas_call(
        body,
        out_shape=jax.ShapeDtypeStruct((M, N), jnp.float32),
        in_specs=[
            pl.BlockSpec(memory_space=pltpu.VMEM),
            pl.BlockSpec(memory_space=pltpu.VMEM),
            pl.BlockSpec(memory_space=pltpu.SMEM),
            pl.BlockSpec(memory_space=pltpu.SMEM),
            pl.BlockSpec(memory_space=pl.ANY),
        ],
        out_specs=pl.BlockSpec(memory_space=pl.ANY),
        input_output_aliases={4: 0},
        scratch_shapes=[
            pltpu.VMEM((2, N_CH, MC, W), jnp.bfloat16),
            pltpu.VMEM((2, N_CH, N_SLOT, MC, W), jnp.bfloat16),
            pltpu.VMEM((MC, W), jnp.float32),
            pltpu.SemaphoreType.DMA((2, N_CH, N_SLOT)),
            pltpu.SemaphoreType.DMA((2, N_CH, N_SLOT)),
            pltpu.SemaphoreType.REGULAR((2, N_CH)),
            pltpu.SemaphoreType.DMA,
        ],
        compiler_params=pltpu.CompilerParams(
            collective_id=0,
            vmem_limit_bytes=64 * 1024 * 1024,
        ),
    )(x, w_mat, scale_x, scale_w, jnp.zeros((M, N), jnp.float32))


# device time: 664690 ns/iter; 1.0759x vs baseline; 1.0759x over previous
import jax
import jax.numpy as jnp
from jax import lax
from jax.experimental import pallas as pl
from jax.experimental.pallas import tpu as pltpu

N_DEV = 4
MC = 1024
W = 1024
N_CH = 2
N_SLOT = 2
N_GRP = 2


def kernel(x, w_mat, scale_x, scale_w):
    M, _ = x.shape
    _, N = w_mat.shape
    x = x.astype(jnp.float8_e5m2)
    w_mat = w_mat.astype(jnp.float8_e5m2)

    def body(x_ref, w_ref, sx_ref, sw_ref, out_ref,
             a_bufs, b_bufs, stage, send_sems, recv_sems, credit_sems,
             store_sem):
        d = lax.axis_index("i")
        left = lax.rem(d + N_DEV - 1, N_DEV)
        right = lax.rem(d + 1, N_DEV)

        barrier_sem = pltpu.get_barrier_semaphore()
        for nbr in (left, right):
            pl.semaphore_signal(barrier_sem, inc=1, device_id=(nbr,),
                                device_id_type=pl.DeviceIdType.MESH)
        pl.semaphore_wait(barrier_sem, 2)

        scale = sx_ref[0] * sw_ref[0]
        dst = (right, left)
        ups = (left, right)

        def pnl(g, ch, di):
            return 4 * g + 2 * ch + di

        def pchunk(c, p):
            xa = x_ref[pl.ds(c * MC, MC), :]
            wb = w_ref[:, pl.ds(p * W, W)]
            return jnp.dot(xa, wb, preferred_element_type=jnp.float32)

        def start_hop(ch, h, srcs, need_credit):
            descs = []
            for di in range(2):
                if need_credit:
                    pl.semaphore_wait(credit_sems.at[di, ch], 1)
                rd = pltpu.make_async_remote_copy(
                    src_ref=srcs[di],
                    dst_ref=b_bufs.at[di, ch, h % N_SLOT],
                    send_sem=send_sems.at[di, ch, h % N_SLOT],
                    recv_sem=recv_sems.at[di, ch, h % N_SLOT],
                    device_id=(dst[di],),
                    device_id_type=pl.DeviceIdType.MESH,
                )
                rd.start()
                descs.append(rd)
            return descs

        def send_credits(ch):
            for di in range(2):
                pl.semaphore_signal(credit_sems.at[di, ch], inc=1,
                                    device_id=(ups[di],),
                                    device_id_type=pl.DeviceIdType.MESH)

        def store(val_f32, c, p):
            stage[...] = val_f32
            st = pltpu.make_async_copy(
                stage, out_ref.at[pl.ds(c * MC, MC), pl.ds(p * W, W)],
                store_sem)
            st.start()
            st.wait()

        own = (lax.rem(d + 1, N_DEV), lax.rem(d + N_DEV - 1, N_DEV))

        infl = {}
        for g in range(N_GRP):
            if g == 0:
                for ch in range(N_CH):
                    a_bufs[0, ch] = pchunk(d, pnl(g, ch, 0)).astype(
                        jnp.bfloat16)
                    a_bufs[1, ch] = pchunk(d, pnl(g, ch, 1)).astype(
                        jnp.bfloat16)
                    infl[ch] = start_hop(
                        ch, 0, (a_bufs.at[0, ch], a_bufs.at[1, ch]),
                        need_credit=False)

            for h in range(6):
                for ch in range(N_CH):
                    if h < 3:
                        nxt = (
                            pchunk(lax.rem(d + N_DEV - h - 1, N_DEV),
                                   pnl(g, ch, 0)),
                            pchunk(lax.rem(d + h + 1, N_DEV),
                                   pnl(g, ch, 1)),
                        )
                    for rd in infl[ch]:
                        rd.wait()

                    slot = h % N_SLOT
                    if h < 2:
                        for di in range(2):
                            a_bufs[di, ch] = (
                                b_bufs[di, ch, slot] + nxt[di]
                            ).astype(jnp.bfloat16)
                        send_credits(ch)
                        infl[ch] = start_hop(
                            ch, h + 1,
                            (a_bufs.at[0, ch], a_bufs.at[1, ch]),
                            need_credit=(6 * g + h + 1 >= 2))
                    elif h == 2:
                        acts = []
                        for di in range(2):
                            y = (b_bufs[di, ch, slot] + nxt[di]) * scale
                            act = y / (1.0 + jnp.exp(-jnp.clip(y, -60.0,
                                                               60.0)))
                            a_bufs[di, ch] = act.astype(jnp.bfloat16)
                            acts.append(act)
                        send_credits(ch)
                        infl[ch] = start_hop(
                            ch, 3, (a_bufs.at[0, ch], a_bufs.at[1, ch]),
                            need_credit=True)
                        for di in range(2):
                            store(acts[di], own[di], pnl(g, ch, di))
                    else:
                        t = h - 3
                        if h == 4:
                            send_credits(ch)
                        if h == 5 and g < N_GRP - 1:
                            send_credits(ch)
                        if h < 5:
                            infl[ch] = start_hop(
                                ch, h + 1,
                                (b_bufs.at[0, ch, slot],
                                 b_bufs.at[1, ch, slot]),
                                need_credit=True)
                        elif g < N_GRP - 1:
                            a_bufs[0, ch] = pchunk(
                                d, pnl(g + 1, ch, 0)).astype(jnp.bfloat16)
                            a_bufs[1, ch] = pchunk(
                                d, pnl(g + 1, ch, 1)).astype(jnp.bfloat16)
                            infl[ch] = start_hop(
                                ch, 0,
                                (a_bufs.at[0, ch], a_bufs.at[1, ch]),
                                need_credit=True)
                        rows = (lax.rem(d + N_DEV - t, N_DEV),
                                lax.rem(d + t, N_DEV))
                        for di in range(2):
                            store(b_bufs[di, ch, slot].astype(jnp.float32),
                                  rows[di], pnl(g, ch, di))
                        if h == 5 and g < N_GRP - 1:
                            send_credits(ch)

    return pl.pallas_call(
        body,
        out_shape=jax.ShapeDtypeStruct((M, N), jnp.float32),
        in_specs=[
            pl.BlockSpec(memory_space=pltpu.VMEM),
            pl.BlockSpec(memory_space=pltpu.VMEM),
            pl.BlockSpec(memory_space=pltpu.SMEM),
            pl.BlockSpec(memory_space=pltpu.SMEM),
        ],
        out_specs=pl.BlockSpec(memory_space=pl.ANY),
        scratch_shapes=[
            pltpu.VMEM((2, N_CH, MC, W), jnp.bfloat16),
            pltpu.VMEM((2, N_CH, N_SLOT, MC, W), jnp.bfloat16),
            pltpu.VMEM((MC, W), jnp.float32),
            pltpu.SemaphoreType.DMA((2, N_CH, N_SLOT)),
            pltpu.SemaphoreType.DMA((2, N_CH, N_SLOT)),
            pltpu.SemaphoreType.REGULAR((2, N_CH)),
            pltpu.SemaphoreType.DMA,
        ],
        compiler_params=pltpu.CompilerParams(
            collective_id=0,
            vmem_limit_bytes=64 * 1024 * 1024,
        ),
    )(x, w_mat, scale_x, scale_w)


# device time: 664630 ns/iter; 1.0759x vs baseline; 1.0001x over previous
import jax
import jax.numpy as jnp
from jax import lax
from jax.experimental import pallas as pl
from jax.experimental.pallas import tpu as pltpu

N_DEV = 4
MC = 1024
W = 1024
N_CH = 2
N_SLOT = 2
N_GRP = 2


def kernel(x, w_mat, scale_x, scale_w):
    M, _ = x.shape
    _, N = w_mat.shape
    x = x.astype(jnp.float8_e5m2)
    w_mat = w_mat.astype(jnp.float8_e5m2)

    def body(x_ref, w_ref, sx_ref, sw_ref, out_ref,
             a_bufs, b_bufs, stage, send_sems, recv_sems, credit_sems,
             store_sem):
        d = lax.axis_index("i")
        left = lax.rem(d + N_DEV - 1, N_DEV)
        right = lax.rem(d + 1, N_DEV)

        barrier_sem = pltpu.get_barrier_semaphore()
        for nbr in (left, right):
            pl.semaphore_signal(barrier_sem, inc=1, device_id=(nbr,),
                                device_id_type=pl.DeviceIdType.MESH)
        pl.semaphore_wait(barrier_sem, 2)

        scale = sx_ref[0] * sw_ref[0]
        dst = (right, left)
        ups = (left, right)

        def pnl(g, ch, di):
            return 4 * g + 2 * ch + di

        def pchunk(c, p):
            xa = x_ref[pl.ds(c * MC, MC), :]
            wb = w_ref[:, pl.ds(p * W, W)]
            return jnp.dot(xa, wb, preferred_element_type=jnp.float32)

        def start_hop(ch, h, srcs, need_credit):
            descs = []
            for di in range(2):
                if need_credit:
                    pl.semaphore_wait(credit_sems.at[di, ch], 1)
                rd = pltpu.make_async_remote_copy(
                    src_ref=srcs[di],
                    dst_ref=b_bufs.at[di, ch, h % N_SLOT],
                    send_sem=send_sems.at[di, ch, h % N_SLOT],
                    recv_sem=recv_sems.at[di, ch, h % N_SLOT],
                    device_id=(dst[di],),
                    device_id_type=pl.DeviceIdType.MESH,
                )
                rd.start()
                descs.append(rd)
            return descs

        def send_credits(ch):
            for di in range(2):
                pl.semaphore_signal(credit_sems.at[di, ch], inc=1,
                                    device_id=(ups[di],),
                                    device_id_type=pl.DeviceIdType.MESH)

        def store(val_f32, c, p):
            stage[...] = val_f32
            st = pltpu.make_async_copy(
                stage, out_ref.at[pl.ds(c * MC, MC), pl.ds(p * W, W)],
                store_sem)
            st.start()
            st.wait()

        own = (lax.rem(d + 1, N_DEV), lax.rem(d + N_DEV - 1, N_DEV))

        infl = {}
        for g in range(N_GRP):
            if g == 0:
                for ch in range(N_CH):
                    a_bufs[0, ch] = pchunk(d, pnl(g, ch, 0)).astype(
                        jnp.bfloat16)
                    a_bufs[1, ch] = pchunk(d, pnl(g, ch, 1)).astype(
                        jnp.bfloat16)
                    infl[ch] = start_hop(
                        ch, 0, (a_bufs.at[0, ch], a_bufs.at[1, ch]),
                        need_credit=False)

            for h in range(6):
                for ch in range(N_CH):
                    if h < 3:
                        nxt = (
                            pchunk(lax.rem(d + N_DEV - h - 1, N_DEV),
                                   pnl(g, ch, 0)),
                            pchunk(lax.rem(d + h + 1, N_DEV),
                                   pnl(g, ch, 1)),
                        )
                    for rd in infl[ch]:
                        rd.wait()

                    slot = h % N_SLOT
                    if h < 2:
                        for di in range(2):
                            a_bufs[di, ch] = (
                                b_bufs[di, ch, slot] + nxt[di]
                            ).astype(jnp.bfloat16)
                        send_credits(ch)
                        infl[ch] = start_hop(
                            ch, h + 1,
                            (a_bufs.at[0, ch], a_bufs.at[1, ch]),
                            need_credit=(6 * g + h + 1 >= 2))
                    elif h == 2:
                        acts = []
                        for di in range(2):
                            y = (b_bufs[di, ch, slot] + nxt[di]) * scale
                            act = y / (1.0 + jnp.exp(-jnp.clip(y, -60.0,
                                                               60.0)))
                            a_bufs[di, ch] = act.astype(jnp.bfloat16)
                            acts.append(act)
                        send_credits(ch)
                        infl[ch] = start_hop(
                            ch, 3, (a_bufs.at[0, ch], a_bufs.at[1, ch]),
                            need_credit=True)
                        for di in range(2):
                            store(acts[di], own[di], pnl(g, ch, di))
                    else:
                        t = h - 3
                        if h == 4:
                            send_credits(ch)
                        if h == 5 and g < N_GRP - 1:
                            send_credits(ch)
                        if h < 5:
                            infl[ch] = start_hop(
                                ch, h + 1,
                                (b_bufs.at[0, ch, slot],
                                 b_bufs.at[1, ch, slot]),
                                need_credit=True)
                        elif g < N_GRP - 1:
                            a_bufs[0, ch] = pchunk(
                                d, pnl(g + 1, ch, 0)).astype(jnp.bfloat16)
                            a_bufs[1, ch] = pchunk(
                                d, pnl(g + 1, ch, 1)).astype(jnp.bfloat16)
                            infl[ch] = start_hop(
                                ch, 0,
                                (a_bufs.at[0, ch], a_bufs.at[1, ch]),
                                need_credit=True)
                        rows = (lax.rem(d + N_DEV - t, N_DEV),
                                lax.rem(d + t, N_DEV))
                        for di in range(2):
                            store(b_bufs[di, ch, slot].astype(jnp.float32),
                                  rows[di], pnl(g, ch, di))
                        if h == 5 and g < N_GRP - 1:
                            send_credits(ch)

    return pl.pallas_call(
        body,
        out_shape=jax.ShapeDtypeStruct((M, N), jnp.float32),
        in_specs=[
            pl.BlockSpec(memory_space=pltpu.VMEM),
            pl.BlockSpec(memory_space=pltpu.VMEM),
            pl.BlockSpec(memory_space=pltpu.SMEM),
            pl.BlockSpec(memory_space=pltpu.SMEM),
        ],
        out_specs=pl.BlockSpec(memory_space=pltpu.MemorySpace.HBM),
        scratch_shapes=[
            pltpu.VMEM((2, N_CH, MC, W), jnp.bfloat16),
            pltpu.VMEM((2, N_CH, N_SLOT, MC, W), jnp.bfloat16),
            pltpu.VMEM((MC, W), jnp.float32),
            pltpu.SemaphoreType.DMA((2, N_CH, N_SLOT)),
            pltpu.SemaphoreType.DMA((2, N_CH, N_SLOT)),
            pltpu.SemaphoreType.REGULAR((2, N_CH)),
            pltpu.SemaphoreType.DMA,
        ],
        compiler_params=pltpu.CompilerParams(
            collective_id=0,
            vmem_limit_bytes=64 * 1024 * 1024,
        ),
    )(x, w_mat, scale_x, scale_w)
